# Initial kernel scaffold; baseline (speedup 1.0000x reference)
import functools

import jax
import jax.numpy as jnp
from jax import lax
from jax.experimental import pallas as pl
from jax.experimental.pallas import tpu as pltpu
from jax.experimental.pallas import tpu_sc as plsc

NC = 2
NS = 16
NW = NC * NS
CH = 128
DEGW = 16


def _sc_mesh():
    return plsc.VectorSubcoreMesh(core_axis_name="c", subcore_axis_name="s")


def _make_deg_kernel(NP, NCH):
    ZR = NP // NS

    @functools.partial(
        pl.kernel,
        out_type=jax.ShapeDtypeStruct((NC, NP, DEGW), jnp.float32),
        mesh=_sc_mesh(),
        scratch_types=[
            pltpu.VMEM_SHARED((NP, DEGW), jnp.float32),
            pltpu.VMEM((NCH, CH), jnp.int32),
            pltpu.VMEM((CH, DEGW), jnp.float32),
            pltpu.VMEM((ZR, DEGW), jnp.float32),
        ],
    )
    def deg_kernel(dst_hbm, out_hbm, acc, dstv, ones, zbuf):
        c = lax.axis_index("c")
        s = lax.axis_index("s")
        wid = c * NS + s

        def fill_ones(i, _):
            ones[i] = jnp.ones((16,), jnp.float32)
            return 0

        lax.fori_loop(0, CH, fill_ones, 0)

        def fill_z(i, _):
            zbuf[i] = jnp.zeros((16,), jnp.float32)
            return 0

        lax.fori_loop(0, ZR, fill_z, 0)

        pltpu.sync_copy(zbuf, acc.at[pl.ds(s * ZR, ZR)])
        pltpu.sync_copy(dst_hbm.at[wid], dstv)
        plsc.subcore_barrier()

        def body(j, _):
            pltpu.sync_copy(ones, acc.at[dstv.at[j]], add=True)
            return 0

        lax.fori_loop(0, NCH, body, 0)
        plsc.subcore_barrier()
        pltpu.sync_copy(acc.at[pl.ds(s * ZR, ZR)],
                        out_hbm.at[c, pl.ds(s * ZR, ZR)])

    return deg_kernel


def _make_layer_kernel(NP, D, NCH):
    RPT = NP // NS

    @functools.partial(
        pl.kernel,
        out_type=jax.ShapeDtypeStruct((NC, NP, D), jnp.float32),
        mesh=_sc_mesh(),
        scratch_types=[
            pltpu.VMEM_SHARED((NP, D), jnp.float32),
            pltpu.VMEM((NCH, CH), jnp.int32),
            pltpu.VMEM((NCH, CH), jnp.int32),
            pltpu.VMEM((CH, D), jnp.float32),
            pltpu.VMEM((CH, D), jnp.float32),
            pltpu.SemaphoreType.DMA,
            pltpu.SemaphoreType.DMA,
        ],
    )
    def layer_kernel(hs_hbm, src_hbm, dst_hbm, out_hbm,
                     acc, srcv, dstv, rows_a, rows_b, sem_a, sem_b):
        c = lax.axis_index("c")
        s = lax.axis_index("s")
        wid = c * NS + s

        pltpu.sync_copy(hs_hbm.at[pl.ds(s * RPT, RPT)],
                        acc.at[pl.ds(s * RPT, RPT)])
        pltpu.sync_copy(src_hbm.at[wid], srcv)
        pltpu.sync_copy(dst_hbm.at[wid], dstv)
        plsc.subcore_barrier()

        def body(j, _):
            ja = 2 * j
            jb = 2 * j + 1
            da = pltpu.async_copy(hs_hbm.at[srcv.at[ja]], rows_a, sem_a)
            db = pltpu.async_copy(hs_hbm.at[srcv.at[jb]], rows_b, sem_b)
            da.wait()
            pltpu.sync_copy(rows_a, acc.at[dstv.at[ja]], add=True)
            db.wait()
            pltpu.sync_copy(rows_b, acc.at[dstv.at[jb]], add=True)
            return 0

        lax.fori_loop(0, NCH // 2, body, 0)
        plsc.subcore_barrier()
        pltpu.sync_copy(acc.at[pl.ds(s * RPT, RPT)],
                        out_hbm.at[c, pl.ds(s * RPT, RPT)])

    return layer_kernel


def _row_block(NP):
    rb = 1024
    while NP % rb:
        rb //= 2
    return rb


def _dinv_of(dega, degb):
    deg = dega[:, 0:1] + degb[:, 0:1] + 1.0
    return lax.rsqrt(deg)


def _make_tc1(NP, D):
    RB = _row_block(NP)

    def body(dega, degb, x, w, o):
        dinv = _dinv_of(dega, degb)
        o[...] = jnp.dot(x[...], w[...],
                         preferred_element_type=jnp.float32) * dinv

    return pl.pallas_call(
        body,
        grid=(NP // RB,),
        in_specs=[
            pl.BlockSpec((RB, DEGW), lambda i: (i, 0)),
            pl.BlockSpec((RB, DEGW), lambda i: (i, 0)),
            pl.BlockSpec((RB, D), lambda i: (i, 0)),
            pl.BlockSpec((D, D), lambda i: (0, 0)),
        ],
        out_specs=pl.BlockSpec((RB, D), lambda i: (i, 0)),
        out_shape=jax.ShapeDtypeStruct((NP, D), jnp.float32),
    )


def _make_tc2(NP, D):
    RB = _row_block(NP)

    def body(dega, degb, acc0, acc1, hs1, b1, w2, o):
        dinv = _dinv_of(dega, degb)
        z = (acc0[...] + acc1[...] - hs1[...]) * dinv + b1[...]
        z = jnp.maximum(z, 0.0)
        o[...] = jnp.dot(z, w2[...],
                         preferred_element_type=jnp.float32) * dinv

    return pl.pallas_call(
        body,
        grid=(NP // RB,),
        in_specs=[
            pl.BlockSpec((RB, DEGW), lambda i: (i, 0)),
            pl.BlockSpec((RB, DEGW), lambda i: (i, 0)),
            pl.BlockSpec((RB, D), lambda i: (i, 0)),
            pl.BlockSpec((RB, D), lambda i: (i, 0)),
            pl.BlockSpec((RB, D), lambda i: (i, 0)),
            pl.BlockSpec((1, D), lambda i: (0, 0)),
            pl.BlockSpec((D, D), lambda i: (0, 0)),
        ],
        out_specs=pl.BlockSpec((RB, D), lambda i: (i, 0)),
        out_shape=jax.ShapeDtypeStruct((NP, D), jnp.float32),
    )


def _make_tc3(NP, D):
    RB = _row_block(NP)

    def body(dega, degb, acc0, acc1, hs2, b2, x0, o):
        dinv = _dinv_of(dega, degb)
        out2 = (acc0[...] + acc1[...] - hs2[...]) * dinv + b2[...]
        h = jnp.maximum(x0[...] + out2, 0.0)
        nrm = jnp.sqrt(jnp.sum(h * h, axis=1, keepdims=True))
        o[...] = h / jnp.maximum(nrm, 1e-12)

    return pl.pallas_call(
        body,
        grid=(NP // RB,),
        in_specs=[
            pl.BlockSpec((RB, DEGW), lambda i: (i, 0)),
            pl.BlockSpec((RB, DEGW), lambda i: (i, 0)),
            pl.BlockSpec((RB, D), lambda i: (i, 0)),
            pl.BlockSpec((RB, D), lambda i: (i, 0)),
            pl.BlockSpec((RB, D), lambda i: (i, 0)),
            pl.BlockSpec((1, D), lambda i: (0, 0)),
            pl.BlockSpec((RB, D), lambda i: (i, 0)),
        ],
        out_specs=pl.BlockSpec((RB, D), lambda i: (i, 0)),
        out_shape=jax.ShapeDtypeStruct((NP, D), jnp.float32),
    )


def kernel(node_feature, edge_index, W1, b1, W2, b2):
    N, D = node_feature.shape
    E = edge_index.shape[1]

    NP = ((N + 1 + NS * CH - 1) // (NS * CH)) * (NS * CH)
    ept = -(-E // NW)
    NCH = -(-ept // CH)
    NCH += NCH % 2
    EP = NW * NCH * CH

    src = edge_index[0]
    dst = edge_index[1]
    srcf = jnp.pad(src, (0, NW * ept - E), constant_values=0)
    dstf = jnp.pad(dst, (0, NW * ept - E), constant_values=N)
    srcp = jnp.pad(srcf.reshape(NW, ept), ((0, 0), (0, NCH * CH - ept)),
                   constant_values=0).reshape(NW, NCH, CH)
    dstp = jnp.pad(dstf.reshape(NW, ept), ((0, 0), (0, NCH * CH - ept)),
                   constant_values=N).reshape(NW, NCH, CH)

    xp = jnp.pad(node_feature, ((0, NP - N), (0, 0)))
    b1r = b1.reshape(1, D)
    b2r = b2.reshape(1, D)

    deg = _make_deg_kernel(NP, NCH)(dstp)
    hs1 = _make_tc1(NP, D)(deg[0], deg[1], xp, W1)
    acc1 = _make_layer_kernel(NP, D, NCH)(hs1, srcp, dstp)
    hs2 = _make_tc2(NP, D)(deg[0], deg[1], acc1[0], acc1[1], hs1, b1r, W2)
    acc2 = _make_layer_kernel(NP, D, NCH)(hs2, srcp, dstp)
    out = _make_tc3(NP, D)(deg[0], deg[1], acc2[0], acc2[1], hs2, b2r, xp)
    return out[:N]


# speedup vs baseline: 9.2651x; 9.2651x over previous
import functools

import jax
import jax.numpy as jnp
from jax import lax
from jax.experimental import pallas as pl
from jax.experimental.pallas import tpu as pltpu
from jax.experimental.pallas import tpu_sc as plsc

NC = 2
NS = 16
NW = NC * NS
CH = 128
DEGW = 16


def _sc_mesh():
    return plsc.VectorSubcoreMesh(core_axis_name="c", subcore_axis_name="s")


def _make_deg_kernel(NP, NCH):
    ZR = NP // NS

    @functools.partial(
        pl.kernel,
        out_type=jax.ShapeDtypeStruct((NC, NP, DEGW), jnp.float32),
        mesh=_sc_mesh(),
        scratch_types=[
            pltpu.VMEM_SHARED((NP, DEGW), jnp.float32),
            pltpu.VMEM((NCH, CH), jnp.int32),
            pltpu.VMEM((CH, DEGW), jnp.float32),
            pltpu.VMEM((ZR, DEGW), jnp.float32),
        ],
    )
    def deg_kernel(dst_hbm, out_hbm, acc, dstv, ones, zbuf):
        c = lax.axis_index("c")
        s = lax.axis_index("s")
        wid = c * NS + s

        def fill_ones(i, _):
            ones[i] = jnp.ones((16,), jnp.float32)
            return 0

        lax.fori_loop(0, CH, fill_ones, 0)

        def fill_z(i, _):
            zbuf[i] = jnp.zeros((16,), jnp.float32)
            return 0

        lax.fori_loop(0, ZR, fill_z, 0)

        pltpu.sync_copy(zbuf, acc.at[pl.ds(s * ZR, ZR)])
        pltpu.sync_copy(dst_hbm.at[wid], dstv)
        plsc.subcore_barrier()

        def body(j, _):
            pltpu.sync_copy(ones, acc.at[dstv.at[j]], add=True)
            return 0

        lax.fori_loop(0, NCH, body, 0)
        plsc.subcore_barrier()
        pltpu.sync_copy(acc.at[pl.ds(s * ZR, ZR)],
                        out_hbm.at[c, pl.ds(s * ZR, ZR)])

    return deg_kernel


def _make_layer_kernel(NP, D, NCH):
    RPT = NP // NS
    IB = 8

    @functools.partial(
        pl.kernel,
        out_type=jax.ShapeDtypeStruct((NC, NP, D), jnp.float32),
        mesh=_sc_mesh(),
        scratch_types=[
            pltpu.VMEM_SHARED((NP, D), jnp.float32),
            pltpu.VMEM((IB, CH), jnp.int32),
            pltpu.VMEM((IB, CH), jnp.int32),
            pltpu.VMEM((CH, D), jnp.float32),
            pltpu.VMEM((CH, D), jnp.float32),
            pltpu.SemaphoreType.DMA,
            pltpu.SemaphoreType.DMA,
        ],
    )
    def layer_kernel(hs_hbm, src_hbm, dst_hbm, out_hbm,
                     acc, srcv, dstv, rows_a, rows_b, sem_a, sem_b):
        c = lax.axis_index("c")
        s = lax.axis_index("s")
        wid = c * NS + s

        pltpu.sync_copy(hs_hbm.at[pl.ds(s * RPT, RPT)],
                        acc.at[pl.ds(s * RPT, RPT)])
        plsc.subcore_barrier()

        def group(g, _):
            pltpu.sync_copy(src_hbm.at[wid, pl.ds(g * IB, IB)], srcv)
            pltpu.sync_copy(dst_hbm.at[wid, pl.ds(g * IB, IB)], dstv)

            def body(j, _):
                ja = 2 * j
                jb = 2 * j + 1
                da = pltpu.async_copy(hs_hbm.at[srcv.at[ja]], rows_a, sem_a)
                db = pltpu.async_copy(hs_hbm.at[srcv.at[jb]], rows_b, sem_b)
                da.wait()
                pltpu.sync_copy(rows_a, acc.at[dstv.at[ja]], add=True)
                db.wait()
                pltpu.sync_copy(rows_b, acc.at[dstv.at[jb]], add=True)
                return 0

            lax.fori_loop(0, IB // 2, body, 0)
            return 0

        lax.fori_loop(0, NCH // IB, group, 0)
        plsc.subcore_barrier()
        pltpu.sync_copy(acc.at[pl.ds(s * RPT, RPT)],
                        out_hbm.at[c, pl.ds(s * RPT, RPT)])

    return layer_kernel


def _row_block(NP):
    rb = 1024
    while NP % rb:
        rb //= 2
    return rb


def _dinv_of(dega, degb):
    deg = dega[:, 0:1] + degb[:, 0:1] + 1.0
    return lax.rsqrt(deg)


def _make_tc1(NP, D):
    RB = _row_block(NP)

    def body(dega, degb, x, w, o):
        dinv = _dinv_of(dega, degb)
        o[...] = jnp.dot(x[...], w[...],
                         preferred_element_type=jnp.float32) * dinv

    return pl.pallas_call(
        body,
        grid=(NP // RB,),
        in_specs=[
            pl.BlockSpec((RB, DEGW), lambda i: (i, 0)),
            pl.BlockSpec((RB, DEGW), lambda i: (i, 0)),
            pl.BlockSpec((RB, D), lambda i: (i, 0)),
            pl.BlockSpec((D, D), lambda i: (0, 0)),
        ],
        out_specs=pl.BlockSpec((RB, D), lambda i: (i, 0)),
        out_shape=jax.ShapeDtypeStruct((NP, D), jnp.float32),
    )


def _make_tc2(NP, D):
    RB = _row_block(NP)

    def body(dega, degb, acc0, acc1, hs1, b1, w2, o):
        dinv = _dinv_of(dega, degb)
        z = (acc0[...] + acc1[...] - hs1[...]) * dinv + b1[...]
        z = jnp.maximum(z, 0.0)
        o[...] = jnp.dot(z, w2[...],
                         preferred_element_type=jnp.float32) * dinv

    return pl.pallas_call(
        body,
        grid=(NP // RB,),
        in_specs=[
            pl.BlockSpec((RB, DEGW), lambda i: (i, 0)),
            pl.BlockSpec((RB, DEGW), lambda i: (i, 0)),
            pl.BlockSpec((RB, D), lambda i: (i, 0)),
            pl.BlockSpec((RB, D), lambda i: (i, 0)),
            pl.BlockSpec((RB, D), lambda i: (i, 0)),
            pl.BlockSpec((1, D), lambda i: (0, 0)),
            pl.BlockSpec((D, D), lambda i: (0, 0)),
        ],
        out_specs=pl.BlockSpec((RB, D), lambda i: (i, 0)),
        out_shape=jax.ShapeDtypeStruct((NP, D), jnp.float32),
    )


def _make_tc3(NP, D):
    RB = _row_block(NP)

    def body(dega, degb, acc0, acc1, hs2, b2, x0, o):
        dinv = _dinv_of(dega, degb)
        out2 = (acc0[...] + acc1[...] - hs2[...]) * dinv + b2[...]
        h = jnp.maximum(x0[...] + out2, 0.0)
        nrm = jnp.sqrt(jnp.sum(h * h, axis=1, keepdims=True))
        o[...] = h / jnp.maximum(nrm, 1e-12)

    return pl.pallas_call(
        body,
        grid=(NP // RB,),
        in_specs=[
            pl.BlockSpec((RB, DEGW), lambda i: (i, 0)),
            pl.BlockSpec((RB, DEGW), lambda i: (i, 0)),
            pl.BlockSpec((RB, D), lambda i: (i, 0)),
            pl.BlockSpec((RB, D), lambda i: (i, 0)),
            pl.BlockSpec((RB, D), lambda i: (i, 0)),
            pl.BlockSpec((1, D), lambda i: (0, 0)),
            pl.BlockSpec((RB, D), lambda i: (i, 0)),
        ],
        out_specs=pl.BlockSpec((RB, D), lambda i: (i, 0)),
        out_shape=jax.ShapeDtypeStruct((NP, D), jnp.float32),
    )


def kernel(node_feature, edge_index, W1, b1, W2, b2):
    N, D = node_feature.shape
    E = edge_index.shape[1]

    NP = ((N + 1 + 1023) // 1024) * 1024
    ept = -(-E // NW)
    NCH = ((-(-ept // CH) + 7) // 8) * 8

    src = edge_index[0]
    dst = edge_index[1]
    srcf = jnp.pad(src, (0, NW * ept - E), constant_values=0)
    dstf = jnp.pad(dst, (0, NW * ept - E), constant_values=N)
    srcp = jnp.pad(srcf.reshape(NW, ept), ((0, 0), (0, NCH * CH - ept)),
                   constant_values=0).reshape(NW, NCH, CH)
    dstp = jnp.pad(dstf.reshape(NW, ept), ((0, 0), (0, NCH * CH - ept)),
                   constant_values=N).reshape(NW, NCH, CH)

    xp = jnp.pad(node_feature, ((0, NP - N), (0, 0)))
    b1r = b1.reshape(1, D)
    b2r = b2.reshape(1, D)

    deg = _make_deg_kernel(NP, NCH)(dstp)
    hs1 = _make_tc1(NP, D)(deg[0], deg[1], xp, W1)
    acc1 = _make_layer_kernel(NP, D, NCH)(hs1, srcp, dstp)
    hs2 = _make_tc2(NP, D)(deg[0], deg[1], acc1[0], acc1[1], hs1, b1r, W2)
    acc2 = _make_layer_kernel(NP, D, NCH)(hs2, srcp, dstp)
    out = _make_tc3(NP, D)(deg[0], deg[1], acc2[0], acc2[1], hs2, b2r, xp)
    return out[:N]


# speedup vs baseline: 9.6524x; 1.0418x over previous
import functools

import jax
import jax.numpy as jnp
from jax import lax
from jax.experimental import pallas as pl
from jax.experimental.pallas import tpu as pltpu
from jax.experimental.pallas import tpu_sc as plsc

NC = 2
NS = 16
NW = NC * NS
CH = 128
GI = 8
DEGW = 128


def _sc_mesh():
    return plsc.VectorSubcoreMesh(core_axis_name="c", subcore_axis_name="s",
                                  num_cores=NC, num_subcores=NS)


def _make_deg_kernel(NP, NCH):
    ZR = NP // NS

    @functools.partial(
        pl.kernel,
        out_type=jax.ShapeDtypeStruct((NC, NP, DEGW), jnp.float32),
        mesh=_sc_mesh(),
        scratch_types=[
            pltpu.VMEM_SHARED((NP, DEGW), jnp.float32),
            pltpu.VMEM((NCH, CH), jnp.int32),
            pltpu.VMEM((CH, DEGW), jnp.float32),
            pltpu.SemaphoreType.DMA,
        ],
    )
    def deg_kernel(dst_hbm, zero_hbm, one_hbm, out_hbm, acc, dstv, ones, sem):
        c = lax.axis_index("c")
        s = lax.axis_index("s")
        wid = c * NS + s

        pltpu.sync_copy(one_hbm, ones)
        pltpu.sync_copy(zero_hbm.at[pl.ds(s * ZR, ZR)],
                        acc.at[pl.ds(s * ZR, ZR)])
        pltpu.sync_copy(dst_hbm.at[wid], dstv)
        plsc.subcore_barrier()

        def body(j, _):
            pltpu.async_copy(ones, acc.at[dstv.at[j]], sem, add=True).wait()
            return 0

        lax.fori_loop(0, NCH, body, 0)
        plsc.subcore_barrier()
        pltpu.sync_copy(acc.at[pl.ds(s * ZR, ZR)],
                        out_hbm.at[c, pl.ds(s * ZR, ZR)])

    return deg_kernel


def _make_layer_kernel(NP, D, NCH):
    RPT = NP // NS
    NG = NCH // GI
    assert NG >= 4 and NG % 2 == 0

    @functools.partial(
        pl.kernel,
        out_type=jax.ShapeDtypeStruct((NC, NP, D), jnp.float32),
        mesh=_sc_mesh(),
        scratch_types=[
            pltpu.VMEM_SHARED((NP, D), jnp.float32),
            pltpu.VMEM((NCH, CH), jnp.int32),
            [pltpu.VMEM((GI, CH), jnp.int32)] * 2,
            [pltpu.VMEM((CH, D), jnp.float32)] * 2,
            [pltpu.SemaphoreType.DMA] * 2,
            [pltpu.SemaphoreType.DMA] * 2,
            [pltpu.SemaphoreType.DMA] * 2,
        ],
    )
    def layer_kernel(hs_hbm, src_hbm, dst_hbm, out_hbm,
                     acc, dstv, srcg, rows, gsem, ssem, psem):
        c = lax.axis_index("c")
        s = lax.axis_index("s")
        wid = c * NS + s

        def prefetch(g, h):
            pltpu.async_copy(src_hbm.at[wid, pl.ds(g * GI, GI)], srcg[h],
                             psem[h])

        def pwait(h):
            pltpu.make_async_copy(src_hbm.at[wid, pl.ds(0, GI)], srcg[h],
                                  psem[h]).wait()

        def gather(j, r, h, b):
            pltpu.async_copy(hs_hbm.at[srcg[h].at[r]], rows[b], gsem[b])

        def gwait(b):
            pltpu.make_async_copy(hs_hbm.at[srcg[0].at[0]], rows[b],
                                  gsem[b]).wait()

        def scatter(j, b):
            pltpu.async_copy(rows[b], acc.at[dstv.at[j]], ssem[b], add=True)

        def swait(b):
            pltpu.make_async_copy(rows[b], acc.at[dstv.at[0]],
                                  ssem[b]).wait()

        pltpu.sync_copy(hs_hbm.at[pl.ds(s * RPT, RPT)],
                        acc.at[pl.ds(s * RPT, RPT)])
        pltpu.sync_copy(dst_hbm.at[wid], dstv)
        prefetch(0, 0)
        prefetch(1, 1)
        plsc.subcore_barrier()

        pwait(0)
        gather(0, 0, 0, 0)
        gather(1, 1, 0, 1)
        gwait(0)
        scatter(0, 0)
        for g in (0, 1):
            for r in range(GI):
                j = g * GI + r
                if j == 0:
                    continue
                b = j % 2
                swait(1 - b)
                if r == GI - 1 and g % 2 == 0:
                    pwait(1)
                elif r == GI - 1:
                    pwait(0)
                gather(j + 1, (r + 1) % GI, (g + (1 if r == GI - 1 else 0)) % 2,
                       1 - b)
                gwait(b)
                if r == GI - 1:
                    prefetch(g + 2, g % 2)
                scatter(j, b)

        def pair(q, _):
            for h in (0, 1):
                g = 2 * q + h
                for r in range(GI):
                    j = g * GI + r
                    b = r % 2
                    swait(1 - b)
                    if r == GI - 1:
                        @pl.when(g + 1 < NG)
                        def _():
                            pwait(1 - h)

                        @pl.when(j + 1 < NCH)
                        def _():
                            gather(j + 1, 0, 1 - h, 1 - b)
                    else:
                        gather(j + 1, r + 1, h, 1 - b)
                    gwait(b)
                    if r == GI - 1:
                        @pl.when(g + 2 < NG)
                        def _():
                            prefetch(g + 2, h)

                    scatter(j, b)
            return 0

        lax.fori_loop(1, NG // 2, pair, 0)
        swait((NCH - 1) % 2)
        plsc.subcore_barrier()
        pltpu.sync_copy(acc.at[pl.ds(s * RPT, RPT)],
                        out_hbm.at[c, pl.ds(s * RPT, RPT)])

    return layer_kernel


def _dinv_of(dega, degb):
    deg = dega[:, 0:1] + degb[:, 0:1] + 1.0
    return lax.rsqrt(deg)


def _make_tc1(NP, D):
    RB = NP // 16

    def body(dega, degb, x, w, o):
        dinv = _dinv_of(dega, degb)
        o[...] = jnp.dot(x[...], w[...],
                         preferred_element_type=jnp.float32) * dinv

    return pl.pallas_call(
        body,
        grid=(NP // RB,),
        in_specs=[
            pl.BlockSpec((RB, DEGW), lambda i: (i, 0)),
            pl.BlockSpec((RB, DEGW), lambda i: (i, 0)),
            pl.BlockSpec((RB, D), lambda i: (i, 0)),
            pl.BlockSpec((D, D), lambda i: (0, 0)),
        ],
        out_specs=pl.BlockSpec((RB, D), lambda i: (i, 0)),
        out_shape=jax.ShapeDtypeStruct((NP, D), jnp.float32),
    )


def _make_tc2(NP, D):
    RB = NP // 16

    def body(dega, degb, acc0, acc1, hs1, b1, w2, o):
        dinv = _dinv_of(dega, degb)
        z = (acc0[...] + acc1[...] - hs1[...]) * dinv + b1[...]
        z = jnp.maximum(z, 0.0)
        o[...] = jnp.dot(z, w2[...],
                         preferred_element_type=jnp.float32) * dinv

    return pl.pallas_call(
        body,
        grid=(NP // RB,),
        in_specs=[
            pl.BlockSpec((RB, DEGW), lambda i: (i, 0)),
            pl.BlockSpec((RB, DEGW), lambda i: (i, 0)),
            pl.BlockSpec((RB, D), lambda i: (i, 0)),
            pl.BlockSpec((RB, D), lambda i: (i, 0)),
            pl.BlockSpec((RB, D), lambda i: (i, 0)),
            pl.BlockSpec((1, D), lambda i: (0, 0)),
            pl.BlockSpec((D, D), lambda i: (0, 0)),
        ],
        out_specs=pl.BlockSpec((RB, D), lambda i: (i, 0)),
        out_shape=jax.ShapeDtypeStruct((NP, D), jnp.float32),
    )


def _make_tc3(NP, D):
    RB = NP // 16

    def body(dega, degb, acc0, acc1, hs2, b2, x0, o):
        dinv = _dinv_of(dega, degb)
        out2 = (acc0[...] + acc1[...] - hs2[...]) * dinv + b2[...]
        h = jnp.maximum(x0[...] + out2, 0.0)
        nrm = jnp.sqrt(jnp.sum(h * h, axis=1, keepdims=True))
        o[...] = h / jnp.maximum(nrm, 1e-12)

    return pl.pallas_call(
        body,
        grid=(NP // RB,),
        in_specs=[
            pl.BlockSpec((RB, DEGW), lambda i: (i, 0)),
            pl.BlockSpec((RB, DEGW), lambda i: (i, 0)),
            pl.BlockSpec((RB, D), lambda i: (i, 0)),
            pl.BlockSpec((RB, D), lambda i: (i, 0)),
            pl.BlockSpec((RB, D), lambda i: (i, 0)),
            pl.BlockSpec((1, D), lambda i: (0, 0)),
            pl.BlockSpec((RB, D), lambda i: (i, 0)),
        ],
        out_specs=pl.BlockSpec((RB, D), lambda i: (i, 0)),
        out_shape=jax.ShapeDtypeStruct((NP, D), jnp.float32),
    )


def kernel(node_feature, edge_index, W1, b1, W2, b2):
    N, D = node_feature.shape
    E = edge_index.shape[1]

    NP = ((N + 1 + 127) // 128) * 128
    ept = -(-E // NW)
    NCH = ((-(-ept // CH) + 2 * GI - 1) // (2 * GI)) * (2 * GI)
    NCH = max(NCH, 4 * GI)

    src = edge_index[0]
    dst = edge_index[1]
    srcf = jnp.pad(src, (0, NW * ept - E), constant_values=0)
    dstf = jnp.pad(dst, (0, NW * ept - E), constant_values=N)
    srcp = jnp.pad(srcf.reshape(NW, ept), ((0, 0), (0, NCH * CH - ept)),
                   constant_values=0).reshape(NW, NCH, CH)
    dstp = jnp.pad(dstf.reshape(NW, ept), ((0, 0), (0, NCH * CH - ept)),
                   constant_values=N).reshape(NW, NCH, CH)

    xp = jnp.pad(node_feature, ((0, NP - N), (0, 0)))
    zeros = jnp.zeros((NP, DEGW), jnp.float32)
    onesa = jnp.ones((CH, DEGW), jnp.float32)
    b1r = b1.reshape(1, D)
    b2r = b2.reshape(1, D)

    deg = _make_deg_kernel(NP, NCH)(dstp, zeros, onesa)
    hs1 = _make_tc1(NP, D)(deg[0], deg[1], xp, W1)
    acc1 = _make_layer_kernel(NP, D, NCH)(hs1, srcp, dstp)
    hs2 = _make_tc2(NP, D)(deg[0], deg[1], acc1[0], acc1[1], hs1, b1r, W2)
    acc2 = _make_layer_kernel(NP, D, NCH)(hs2, srcp, dstp)
    out = _make_tc3(NP, D)(deg[0], deg[1], acc2[0], acc2[1], hs2, b2r, xp)
    return out[:N]


# speedup vs baseline: 9.6590x; 1.0007x over previous
import functools

import jax
import jax.numpy as jnp
from jax import lax
from jax.experimental import pallas as pl
from jax.experimental.pallas import tpu as pltpu
from jax.experimental.pallas import tpu_sc as plsc

NC = 2
NS = 16
NW = NC * NS
CH = 128
GI = 8
DEGW = 128


def _sc_mesh():
    return plsc.VectorSubcoreMesh(core_axis_name="c", subcore_axis_name="s",
                                  num_cores=NC, num_subcores=NS)


def _make_deg_kernel(NP, NCH):
    ZR = NP // NS

    @functools.partial(
        pl.kernel,
        out_type=jax.ShapeDtypeStruct((NC, NP, DEGW), jnp.float32),
        mesh=_sc_mesh(),
        scratch_types=[
            pltpu.VMEM_SHARED((NP, DEGW), jnp.float32),
            pltpu.VMEM((NCH, CH), jnp.int32),
            pltpu.VMEM((CH, DEGW), jnp.float32),
            pltpu.SemaphoreType.DMA,
        ],
    )
    def deg_kernel(dst_hbm, zero_hbm, one_hbm, out_hbm, acc, dstv, ones, sem):
        c = lax.axis_index("c")
        s = lax.axis_index("s")
        wid = c * NS + s

        pltpu.sync_copy(one_hbm, ones)
        pltpu.sync_copy(zero_hbm.at[pl.ds(s * ZR, ZR)],
                        acc.at[pl.ds(s * ZR, ZR)])
        pltpu.sync_copy(dst_hbm.at[wid], dstv)
        plsc.subcore_barrier()

        LAG = 4
        for k in range(LAG):
            pltpu.async_copy(ones, acc.at[dstv.at[k]], sem, add=True)

        def body(j, _):
            pltpu.make_async_copy(ones, acc.at[dstv.at[0]], sem).wait()
            pltpu.async_copy(ones, acc.at[dstv.at[j + LAG]], sem, add=True)
            return 0

        lax.fori_loop(0, NCH - LAG, body, 0)
        for k in range(LAG):
            pltpu.make_async_copy(ones, acc.at[dstv.at[0]], sem).wait()
        plsc.subcore_barrier()
        pltpu.sync_copy(acc.at[pl.ds(s * ZR, ZR)],
                        out_hbm.at[c, pl.ds(s * ZR, ZR)])

    return deg_kernel


def _make_layer_kernel(NP, D, NCH):
    RPT = NP // NS
    NG = NCH // GI
    assert NG >= 4 and NG % 2 == 0

    @functools.partial(
        pl.kernel,
        out_type=jax.ShapeDtypeStruct((NC, NP, D), jnp.float32),
        mesh=_sc_mesh(),
        scratch_types=[
            pltpu.VMEM_SHARED((NP, D), jnp.float32),
            pltpu.VMEM((NCH, CH), jnp.int32),
            [pltpu.VMEM((GI, CH), jnp.int32)] * 2,
            [pltpu.VMEM((CH, D), jnp.float32)] * 2,
            [pltpu.SemaphoreType.DMA] * 2,
            [pltpu.SemaphoreType.DMA] * 2,
            [pltpu.SemaphoreType.DMA] * 2,
        ],
    )
    def layer_kernel(hs_hbm, src_hbm, dst_hbm, out_hbm,
                     acc, dstv, srcg, rows, gsem, ssem, psem):
        c = lax.axis_index("c")
        s = lax.axis_index("s")
        wid = c * NS + s

        def prefetch(g, h):
            pltpu.async_copy(src_hbm.at[wid, pl.ds(g * GI, GI)], srcg[h],
                             psem[h])

        def pwait(h):
            pltpu.make_async_copy(src_hbm.at[wid, pl.ds(0, GI)], srcg[h],
                                  psem[h]).wait()

        def gather(j, r, h, b):
            pltpu.async_copy(hs_hbm.at[srcg[h].at[r]], rows[b], gsem[b])

        def gwait(b):
            pltpu.make_async_copy(hs_hbm.at[srcg[0].at[0]], rows[b],
                                  gsem[b]).wait()

        def scatter(j, b):
            pltpu.async_copy(rows[b], acc.at[dstv.at[j]], ssem[b], add=True)

        def swait(b):
            pltpu.make_async_copy(rows[b], acc.at[dstv.at[0]],
                                  ssem[b]).wait()

        pltpu.sync_copy(hs_hbm.at[pl.ds(s * RPT, RPT)],
                        acc.at[pl.ds(s * RPT, RPT)])
        pltpu.sync_copy(dst_hbm.at[wid], dstv)
        prefetch(0, 0)
        prefetch(1, 1)
        plsc.subcore_barrier()

        pwait(0)
        gather(0, 0, 0, 0)
        gather(1, 1, 0, 1)
        gwait(0)
        scatter(0, 0)
        for g in (0, 1):
            for r in range(GI):
                j = g * GI + r
                if j == 0:
                    continue
                b = j % 2
                swait(1 - b)
                if r == GI - 1 and g % 2 == 0:
                    pwait(1)
                elif r == GI - 1:
                    pwait(0)
                gather(j + 1, (r + 1) % GI, (g + (1 if r == GI - 1 else 0)) % 2,
                       1 - b)
                gwait(b)
                if r == GI - 1:
                    prefetch(g + 2, g % 2)
                scatter(j, b)

        def pair(q, _):
            for h in (0, 1):
                g = 2 * q + h
                for r in range(GI):
                    j = g * GI + r
                    b = r % 2
                    swait(1 - b)
                    if r == GI - 1:
                        @pl.when(g + 1 < NG)
                        def _():
                            pwait(1 - h)

                        @pl.when(j + 1 < NCH)
                        def _():
                            gather(j + 1, 0, 1 - h, 1 - b)
                    else:
                        gather(j + 1, r + 1, h, 1 - b)
                    gwait(b)
                    if r == GI - 1:
                        @pl.when(g + 2 < NG)
                        def _():
                            prefetch(g + 2, h)

                    scatter(j, b)
            return 0

        lax.fori_loop(1, NG // 2, pair, 0)
        swait((NCH - 1) % 2)
        plsc.subcore_barrier()
        pltpu.sync_copy(acc.at[pl.ds(s * RPT, RPT)],
                        out_hbm.at[c, pl.ds(s * RPT, RPT)])

    return layer_kernel


def _dinv_of(dega, degb):
    deg = dega[:, 0:1] + degb[:, 0:1] + 1.0
    return lax.rsqrt(deg)


def _make_tc1(NP, D):
    RB = NP // 16

    def body(dega, degb, x, w, o):
        dinv = _dinv_of(dega, degb)
        o[...] = jnp.dot(x[...], w[...],
                         preferred_element_type=jnp.float32) * dinv

    return pl.pallas_call(
        body,
        grid=(NP // RB,),
        in_specs=[
            pl.BlockSpec((RB, DEGW), lambda i: (i, 0)),
            pl.BlockSpec((RB, DEGW), lambda i: (i, 0)),
            pl.BlockSpec((RB, D), lambda i: (i, 0)),
            pl.BlockSpec((D, D), lambda i: (0, 0)),
        ],
        out_specs=pl.BlockSpec((RB, D), lambda i: (i, 0)),
        out_shape=jax.ShapeDtypeStruct((NP, D), jnp.float32),
    )


def _make_tc2(NP, D):
    RB = NP // 16

    def body(dega, degb, acc0, acc1, hs1, b1, w2, o):
        dinv = _dinv_of(dega, degb)
        z = (acc0[...] + acc1[...] - hs1[...]) * dinv + b1[...]
        z = jnp.maximum(z, 0.0)
        o[...] = jnp.dot(z, w2[...],
                         preferred_element_type=jnp.float32) * dinv

    return pl.pallas_call(
        body,
        grid=(NP // RB,),
        in_specs=[
            pl.BlockSpec((RB, DEGW), lambda i: (i, 0)),
            pl.BlockSpec((RB, DEGW), lambda i: (i, 0)),
            pl.BlockSpec((RB, D), lambda i: (i, 0)),
            pl.BlockSpec((RB, D), lambda i: (i, 0)),
            pl.BlockSpec((RB, D), lambda i: (i, 0)),
            pl.BlockSpec((1, D), lambda i: (0, 0)),
            pl.BlockSpec((D, D), lambda i: (0, 0)),
        ],
        out_specs=pl.BlockSpec((RB, D), lambda i: (i, 0)),
        out_shape=jax.ShapeDtypeStruct((NP, D), jnp.float32),
    )


def _make_tc3(NP, D):
    RB = NP // 16

    def body(dega, degb, acc0, acc1, hs2, b2, x0, o):
        dinv = _dinv_of(dega, degb)
        out2 = (acc0[...] + acc1[...] - hs2[...]) * dinv + b2[...]
        h = jnp.maximum(x0[...] + out2, 0.0)
        nrm = jnp.sqrt(jnp.sum(h * h, axis=1, keepdims=True))
        o[...] = h / jnp.maximum(nrm, 1e-12)

    return pl.pallas_call(
        body,
        grid=(NP // RB,),
        in_specs=[
            pl.BlockSpec((RB, DEGW), lambda i: (i, 0)),
            pl.BlockSpec((RB, DEGW), lambda i: (i, 0)),
            pl.BlockSpec((RB, D), lambda i: (i, 0)),
            pl.BlockSpec((RB, D), lambda i: (i, 0)),
            pl.BlockSpec((RB, D), lambda i: (i, 0)),
            pl.BlockSpec((1, D), lambda i: (0, 0)),
            pl.BlockSpec((RB, D), lambda i: (i, 0)),
        ],
        out_specs=pl.BlockSpec((RB, D), lambda i: (i, 0)),
        out_shape=jax.ShapeDtypeStruct((NP, D), jnp.float32),
    )


def kernel(node_feature, edge_index, W1, b1, W2, b2):
    N, D = node_feature.shape
    E = edge_index.shape[1]

    NP = ((N + 1 + 127) // 128) * 128
    ept = -(-E // NW)
    NCH = ((-(-ept // CH) + 2 * GI - 1) // (2 * GI)) * (2 * GI)
    NCH = max(NCH, 4 * GI)

    src = edge_index[0]
    dst = edge_index[1]
    srcf = jnp.pad(src, (0, NW * ept - E), constant_values=0)
    dstf = jnp.pad(dst, (0, NW * ept - E), constant_values=N)
    srcp = jnp.pad(srcf.reshape(NW, ept), ((0, 0), (0, NCH * CH - ept)),
                   constant_values=0).reshape(NW, NCH, CH)
    dstp = jnp.pad(dstf.reshape(NW, ept), ((0, 0), (0, NCH * CH - ept)),
                   constant_values=N).reshape(NW, NCH, CH)

    xp = jnp.pad(node_feature, ((0, NP - N), (0, 0)))
    zeros = jnp.zeros((NP, DEGW), jnp.float32)
    onesa = jnp.ones((CH, DEGW), jnp.float32)
    b1r = b1.reshape(1, D)
    b2r = b2.reshape(1, D)

    deg = _make_deg_kernel(NP, NCH)(dstp, zeros, onesa)
    hs1 = _make_tc1(NP, D)(deg[0], deg[1], xp, W1)
    acc1 = _make_layer_kernel(NP, D, NCH)(hs1, srcp, dstp)
    hs2 = _make_tc2(NP, D)(deg[0], deg[1], acc1[0], acc1[1], hs1, b1r, W2)
    acc2 = _make_layer_kernel(NP, D, NCH)(hs2, srcp, dstp)
    out = _make_tc3(NP, D)(deg[0], deg[1], acc2[0], acc2[1], hs2, b2r, xp)
    return out[:N]
